# baseline (speedup 1.0000x reference)
import jax
import jax.numpy as jnp
from jax.experimental import pallas as pl

N = 16384
STRIDE = 4
NSAMPLE = 16
EPS = 1e-5


def _fps(points, m):
    n = points.shape[0]

    def body(i, state):
        dists, idxs = state
        last = idxs[i - 1]
        d = jnp.sum((points - points[last]) ** 2, axis=1)
        dists = jnp.minimum(dists, d)
        nxt = jnp.argmax(dists).astype(jnp.int32)
        idxs = idxs.at[i].set(nxt)
        return (dists, idxs)

    dists0 = jnp.full((n,), jnp.inf, dtype=jnp.float32)
    idxs0 = jnp.zeros((m,), dtype=jnp.int32)
    _, idxs = jax.lax.fori_loop(1, m, body, (dists0, idxs0))
    return idxs


def _ident(x_ref, o_ref):
    o_ref[...] = x_ref[...]


def kernel(point, feat, row_splits, W, gamma, beta):
    m = N // STRIDE
    fps_idx = _fps(point, m)
    new_point = point[fps_idx]
    d2 = (jnp.sum(new_point * new_point, axis=1, keepdims=True)
          - 2.0 * (new_point @ point.T)
          + jnp.sum(point * point, axis=1)[None, :])
    _, knn_idx = jax.lax.top_k(-d2, NSAMPLE)
    grouped_xyz = point[knn_idx] - new_point[:, None, :]
    grouped_feat = feat[knn_idx]
    g = jnp.concatenate([grouped_xyz, grouped_feat], axis=-1)
    h = jnp.einsum('mnc,oc->mno', g, W)
    h = jnp.transpose(h, (0, 2, 1))
    mean = jnp.mean(h, axis=(0, 2))
    var = jnp.var(h, axis=(0, 2))
    h = (h - mean[None, :, None]) / jnp.sqrt(var[None, :, None] + EPS)
    h = h * gamma[None, :, None] + beta[None, :, None]
    h = jnp.maximum(h, 0.0)
    out_feat = jnp.max(h, axis=2)
    out_feat = pl.pallas_call(
        _ident, out_shape=jax.ShapeDtypeStruct(out_feat.shape, out_feat.dtype),
    )(out_feat)
    new_row_splits = jnp.array([0, m], dtype=row_splits.dtype)
    return (new_point, out_feat, new_row_splits)


# speedup vs baseline: 5.9982x; 5.9982x over previous
import functools

import jax
import jax.numpy as jnp
from jax import lax
from jax.experimental import pallas as pl
from jax.experimental.pallas import tpu as pltpu
from jax.experimental.pallas import tpu_sc as plsc

N = 16384
IN_PLANES = 256
OUT_PLANES = 512
STRIDE = 4
NSAMPLE = 16
EPS = 1e-5
M = N // STRIDE
R, C = 128, 128
MR = 32
D_PAD = 384
NW = 32
QPW = M // NW
GPW = (M * NSAMPLE) // NW
GCHUNK = 256
BM = 128
GRID4 = M // BM


def _fps_kernel(xs_ref, ys_ref, zs_ref, qx_ref, qy_ref, qz_ref, pn_ref):
    iota = (lax.broadcasted_iota(jnp.int32, (R, C), 0) * C
            + lax.broadcasted_iota(jnp.int32, (R, C), 1))
    miota = (lax.broadcasted_iota(jnp.int32, (MR, C), 0) * C
             + lax.broadcasted_iota(jnp.int32, (MR, C), 1))
    xs = xs_ref[...]
    ys = ys_ref[...]
    zs = zs_ref[...]
    pn_ref[...] = xs * xs + ys * ys + zs * zs

    def body(i, state):
        dists, cur, qx, qy, qz = state
        m2 = iota == cur
        lx = jnp.sum(jnp.where(m2, xs, 0.0))
        ly = jnp.sum(jnp.where(m2, ys, 0.0))
        lz = jnp.sum(jnp.where(m2, zs, 0.0))
        mq = miota == (i - 1)
        qx = jnp.where(mq, lx, qx)
        qy = jnp.where(mq, ly, qy)
        qz = jnp.where(mq, lz, qz)
        dx = xs - lx
        dy = ys - ly
        dz = zs - lz
        d = dx * dx + dy * dy + dz * dz
        dists = jnp.minimum(dists, d)
        mx = jnp.max(dists)
        cand = jnp.where(dists == mx, iota, jnp.int32(N))
        cur = jnp.min(cand)
        return (dists, cur, qx, qy, qz)

    dists0 = jnp.full((R, C), jnp.inf, dtype=jnp.float32)
    q0 = jnp.zeros((MR, C), dtype=jnp.float32)
    _, cur, qx, qy, qz = lax.fori_loop(
        1, M, body, (dists0, jnp.int32(0), q0, q0, q0))
    m2 = iota == cur
    lx = jnp.sum(jnp.where(m2, xs, 0.0))
    ly = jnp.sum(jnp.where(m2, ys, 0.0))
    lz = jnp.sum(jnp.where(m2, zs, 0.0))
    mq = miota == (M - 1)
    qx_ref[...] = jnp.where(mq, lx, qx)
    qy_ref[...] = jnp.where(mq, ly, qy)
    qz_ref[...] = jnp.where(mq, lz, qz)


def _run_fps(point):
    xs = point[:, 0].reshape(R, C)
    ys = point[:, 1].reshape(R, C)
    zs = point[:, 2].reshape(R, C)
    qx, qy, qz, pn = pl.pallas_call(
        _fps_kernel,
        out_shape=[jax.ShapeDtypeStruct((MR, C), jnp.float32)] * 3
        + [jax.ShapeDtypeStruct((R, C), jnp.float32)],
    )(xs, ys, zs)
    return qx.reshape(M), qy.reshape(M), qz.reshape(M), pn.reshape(N)


def _lane_splat(vec, lane_idx):
    idx = jnp.broadcast_to(jnp.int32(lane_idx), (16,))
    dn = lax.GatherDimensionNumbers(
        offset_dims=(), collapsed_slice_dims=(0,), start_index_map=(0,))
    return lax.gather(vec, idx[:, None], dn, slice_sizes=(1,),
                      mode=lax.GatherScatterMode.PROMISE_IN_BOUNDS)
def _d2_kernel(q8_ref, p8t_ref, qn_ref, pn_ref, out_ref):
    dot = jnp.dot(q8_ref[...], p8t_ref[...],
                  preferred_element_type=jnp.float32)
    out_ref[...] = qn_ref[...] - 2.0 * dot + pn_ref[...]


def _run_d2(q8, p8t, qn, pn):
    bq, bp = 512, 2048
    return pl.pallas_call(
        _d2_kernel,
        grid=(M // bq, N // bp),
        in_specs=[
            pl.BlockSpec((bq, 8), lambda i, j: (i, 0)),
            pl.BlockSpec((8, bp), lambda i, j: (0, j)),
            pl.BlockSpec((bq, 1), lambda i, j: (i, 0)),
            pl.BlockSpec((1, bp), lambda i, j: (0, j)),
        ],
        out_specs=pl.BlockSpec((bq, bp), lambda i, j: (i, j)),
        out_shape=jax.ShapeDtypeStruct((M, N), jnp.float32),
    )(q8, p8t, qn, pn)


def _knn_body(d2_h, out_h, buf0_v, buf1_v, out_v, sem0, sem1):
    wid = lax.axis_index("s") * 2 + lax.axis_index("c")
    base = wid * QPW
    lane = lax.iota(jnp.int32, 16)
    bufs = (buf0_v, buf1_v)
    sems = (sem0, sem1)
    pltpu.async_copy(d2_h.at[base], buf0_v, sem0)

    def scan_row(buf):
        def scan_chunk(j, carry):
            topv, topi, tv = carry
            off = j * 16
            d = buf[pl.ds(off, 16)]
            hit = jnp.any(d < tv)

            def merge(args):
                topv, topi, _ = args
                av, ai = plsc.sort_key_val(d, off + lane)
                sv = jnp.flip(av)
                si = jnp.flip(ai)
                take = sv < topv
                mv = jnp.where(take, sv, topv)
                mi = jnp.where(take, si, topi)
                nv, ni = plsc.sort_key_val(mv, mi)
                return nv, ni, _lane_splat(nv, 15)

            return lax.cond(hit, merge, lambda a: a, (topv, topi, tv))

        topv0 = jnp.full((16,), jnp.inf, dtype=jnp.float32)
        topi0 = jnp.zeros((16,), dtype=jnp.int32)
        _, topi, _ = lax.fori_loop(0, N // 16, scan_chunk,
                                   (topv0, topi0, topv0))
        return topi

    def per_pair(k, _):
        for b in range(2):
            q = 2 * k + b
            pltpu.make_async_copy(d2_h.at[base], bufs[b], sems[b]).wait()
            nxt = jnp.minimum(base + q + 1, M - 1)
            pltpu.async_copy(d2_h.at[nxt], bufs[1 - b], sems[1 - b])
            out_v[pl.ds(q * NSAMPLE, NSAMPLE)] = scan_row(bufs[b])
        return 0

    lax.fori_loop(0, QPW // 2, per_pair, 0)
    pltpu.make_async_copy(d2_h.at[base], buf0_v, sem0).wait()
    pltpu.sync_copy(out_v, out_h.at[pl.ds(base * NSAMPLE, QPW * NSAMPLE)])


def _run_knn(d2):
    mesh = plsc.VectorSubcoreMesh(core_axis_name="c", subcore_axis_name="s")
    knn = functools.partial(
        pl.kernel, mesh=mesh,
        compiler_params=pltpu.CompilerParams(needs_layout_passes=False),
        out_type=jax.ShapeDtypeStruct((M * NSAMPLE,), jnp.int32),
        scratch_types=[
            pltpu.VMEM((N,), jnp.float32),
            pltpu.VMEM((N,), jnp.float32),
            pltpu.VMEM((QPW * NSAMPLE,), jnp.int32),
            pltpu.SemaphoreType.DMA,
            pltpu.SemaphoreType.DMA,
        ],
    )(_knn_body)
    return knn(d2)


def _gather_body(tab_h, idx_h, out_h, idx_v, buf_v, sem):
    wid = lax.axis_index("s") * 2 + lax.axis_index("c")
    base = wid * GPW
    pltpu.sync_copy(idx_h.at[pl.ds(base, GPW)], idx_v)

    def chunk(c, _):
        pltpu.async_copy(
            tab_h.at[idx_v.at[pl.ds(c * GCHUNK, GCHUNK)]], buf_v, sem).wait()
        pltpu.sync_copy(buf_v, out_h.at[pl.ds(base + c * GCHUNK, GCHUNK)])
        return 0

    lax.fori_loop(0, GPW // GCHUNK, chunk, 0)


def _run_gather(tab, flat_idx):
    mesh = plsc.VectorSubcoreMesh(core_axis_name="c", subcore_axis_name="s")
    g = functools.partial(
        pl.kernel, mesh=mesh,
        compiler_params=pltpu.CompilerParams(needs_layout_passes=False),
        out_type=jax.ShapeDtypeStruct((M * NSAMPLE, D_PAD), jnp.float32),
        scratch_types=[
            pltpu.VMEM((GPW,), jnp.int32),
            pltpu.VMEM((GCHUNK, D_PAD), jnp.float32),
            pltpu.SemaphoreType.DMA,
        ],
    )(_gather_body)
    return g(tab, flat_idx)


def _mlp_kernel(g_ref, np8_ref, wp_ref, wx_ref,
                maxh_ref, minh_ref, ssum_ref, ssq_ref):
    i = pl.program_id(0)
    g = g_ref[...]
    h2 = jnp.dot(g, wp_ref[...], preferred_element_type=jnp.float32)
    qw = jnp.dot(np8_ref[...], wx_ref[...],
                 preferred_element_type=jnp.float32)
    h3 = h2.reshape(BM, NSAMPLE, OUT_PLANES) - qw[:, None, :]
    maxh_ref[...] = jnp.max(h3, axis=1)
    minh_ref[...] = jnp.min(h3, axis=1)
    flat = h3.reshape(BM * NSAMPLE // 8, 8, OUT_PLANES)
    psum = jnp.sum(flat, axis=0)
    psq = jnp.sum(flat * flat, axis=0)

    @pl.when(i == 0)
    def _():
        ssum_ref[...] = jnp.zeros_like(ssum_ref)
        ssq_ref[...] = jnp.zeros_like(ssq_ref)

    ssum_ref[...] += psum
    ssq_ref[...] += psq


def _run_mlp(gathered, np8, wp, wx8):
    return pl.pallas_call(
        _mlp_kernel,
        grid=(GRID4,),
        in_specs=[
            pl.BlockSpec((BM * NSAMPLE, D_PAD), lambda i: (i, 0)),
            pl.BlockSpec((BM, 8), lambda i: (i, 0)),
            pl.BlockSpec((D_PAD, OUT_PLANES), lambda i: (0, 0)),
            pl.BlockSpec((8, OUT_PLANES), lambda i: (0, 0)),
        ],
        out_specs=[
            pl.BlockSpec((BM, OUT_PLANES), lambda i: (i, 0)),
            pl.BlockSpec((BM, OUT_PLANES), lambda i: (i, 0)),
            pl.BlockSpec((8, OUT_PLANES), lambda i: (0, 0)),
            pl.BlockSpec((8, OUT_PLANES), lambda i: (0, 0)),
        ],
        out_shape=[
            jax.ShapeDtypeStruct((M, OUT_PLANES), jnp.float32),
            jax.ShapeDtypeStruct((M, OUT_PLANES), jnp.float32),
            jax.ShapeDtypeStruct((8, OUT_PLANES), jnp.float32),
            jax.ShapeDtypeStruct((8, OUT_PLANES), jnp.float32),
        ],
    )(gathered, np8, wp, wx8)


def _bn_kernel(maxh_ref, minh_ref, ssum_ref, ssq_ref, gam_ref, bet_ref,
               out_ref):
    cnt = float(M * NSAMPLE)
    mean = jnp.sum(ssum_ref[...], axis=0, keepdims=True) * (1.0 / cnt)
    ex2 = jnp.sum(ssq_ref[...], axis=0, keepdims=True) * (1.0 / cnt)
    var = ex2 - mean * mean
    scale = 1.0 / jnp.sqrt(var + EPS)
    gam = gam_ref[...]
    a = gam * scale
    b = bet_ref[...] - mean * a
    vmax = jnp.maximum(maxh_ref[...] * a + b, 0.0)
    vmin = jnp.maximum(minh_ref[...] * a + b, 0.0)
    out_ref[...] = jnp.where(gam >= 0.0, vmax, vmin)


def _run_bn(maxh, minh, ssum, ssq, gamma2, beta2):
    return pl.pallas_call(
        _bn_kernel,
        grid=(GRID4,),
        in_specs=[
            pl.BlockSpec((BM, OUT_PLANES), lambda i: (i, 0)),
            pl.BlockSpec((BM, OUT_PLANES), lambda i: (i, 0)),
            pl.BlockSpec((8, OUT_PLANES), lambda i: (0, 0)),
            pl.BlockSpec((8, OUT_PLANES), lambda i: (0, 0)),
            pl.BlockSpec((1, OUT_PLANES), lambda i: (0, 0)),
            pl.BlockSpec((1, OUT_PLANES), lambda i: (0, 0)),
        ],
        out_specs=pl.BlockSpec((BM, OUT_PLANES), lambda i: (i, 0)),
        out_shape=jax.ShapeDtypeStruct((M, OUT_PLANES), jnp.float32),
    )(maxh, minh, ssum, ssq, gamma2, beta2)


def kernel(point, feat, row_splits, W, gamma, beta):
    qx, qy, qz, _ = _run_fps(point)
    new_point = jnp.stack([qx, qy, qz], axis=-1)
    q8 = jnp.concatenate([new_point, jnp.zeros((M, 5), jnp.float32)], axis=1)
    p8t = jnp.concatenate([point.T, jnp.zeros((5, N), jnp.float32)], axis=0)
    qn = jnp.sum(new_point * new_point, axis=1, keepdims=True)
    pn2 = jnp.sum(point * point, axis=1)[None, :]
    d2 = _run_d2(q8, p8t, qn, pn2)
    flat_idx = _run_knn(d2)

    tab = jnp.concatenate(
        [point, feat, jnp.zeros((N, D_PAD - 3 - IN_PLANES), jnp.float32)],
        axis=1)
    gathered = _run_gather(tab, flat_idx)

    wp = jnp.concatenate(
        [W.T, jnp.zeros((D_PAD - 3 - IN_PLANES, OUT_PLANES), jnp.float32)],
        axis=0)
    wx8 = jnp.concatenate([W.T[:3], jnp.zeros((5, OUT_PLANES), jnp.float32)],
                          axis=0)
    maxh, minh, ssum, ssq = _run_mlp(gathered, q8, wp, wx8)
    out_feat = _run_bn(maxh, minh, ssum, ssq,
                       gamma.reshape(1, OUT_PLANES), beta.reshape(1, OUT_PLANES))
    new_row_splits = jnp.array([0, M], dtype=row_splits.dtype)
    return (new_point, out_feat, new_row_splits)
